# baseline (speedup 1.0000x reference)
import jax
import jax.numpy as jnp
from jax import lax
from jax.experimental import pallas as pl
from jax.experimental.pallas import tpu as pltpu
from jax.experimental.pallas import tpu_sc as plsc

N = 10000
D = 128
DH = 64
NUM_MSG = 10000
NC = 2
NS = 16
L = 16

HB = 160
CK = 80
MPT = 640
NHP = 10240
RT = 640


def _body(m_hbm, nodes2_hbm, out_hbm,
          sums_sh, cnts_sh,
          rows4, lidx4, ones_v, z16_v, zseed,
          sem_ldn, sem_ldr, sem_scr, sem_sco, sem_wr, sem_rd, sem_zs, sem_zc):
    core = lax.axis_index("c")
    sub = lax.axis_index("s")
    zvec = jnp.zeros((L,), jnp.float32)
    onevec = jnp.ones((L,), jnp.float32)
    cb = core * DH
    rbase = sub * RT
    mbase = sub * MPT
    bbase = sub * (MPT // CK)

    def fire_load(step):
        mb = jnp.minimum(mbase + step * HB, NUM_MSG - HB)
        blk = jnp.minimum(bbase + step * (HB // CK),
                          NUM_MSG // CK - HB // CK)
        dn = pltpu.async_copy(nodes2_hbm.at[pl.ds(blk, HB // CK)],
                              lidx4.at[step], sem_ldn)
        dr = pltpu.async_copy(m_hbm.at[pl.ds(mb, HB), pl.ds(cb, DH)],
                              rows4.at[step], sem_ldr)
        return dn, dr

    lds = [fire_load(0), fire_load(1), fire_load(2), fire_load(3)]

    for i in range(CK):
        ones_v[i, :] = onevec

    def _seed(i, _):
        for g in range(DH // L):
            zseed[i, pl.ds(g * L, L)] = zvec
        z16_v[i, :] = zvec
        return 0
    lax.fori_loop(0, HB, _seed, 0)

    dz = pltpu.async_copy(zseed, sums_sh.at[pl.ds(rbase, HB)], sem_zs)
    dc = pltpu.async_copy(z16_v.at[pl.ds(0, HB)],
                          cnts_sh.at[pl.ds(rbase, HB)], sem_zc)
    dz.wait()
    dc.wait()
    dz = pltpu.async_copy(sums_sh.at[pl.ds(rbase, HB)],
                          sums_sh.at[pl.ds(rbase + HB, HB)], sem_zs)
    dc = pltpu.async_copy(cnts_sh.at[pl.ds(rbase, HB)],
                          cnts_sh.at[pl.ds(rbase + HB, HB)], sem_zc)
    dz.wait()
    dc.wait()
    dz = pltpu.async_copy(sums_sh.at[pl.ds(rbase, 2 * HB)],
                          sums_sh.at[pl.ds(rbase + 2 * HB, 2 * HB)], sem_zs)
    dc = pltpu.async_copy(cnts_sh.at[pl.ds(rbase, 2 * HB)],
                          cnts_sh.at[pl.ds(rbase + 2 * HB, 2 * HB)], sem_zc)
    dz.wait()
    dc.wait()
    plsc.subcore_barrier()

    def fire_scatter(step, nchunks):
        ds = []
        for j in range(nchunks):
            ds.append(pltpu.async_copy(rows4.at[step, pl.ds(j * CK, CK)],
                                       sums_sh.at[lidx4.at[step, j]],
                                       sem_scr, add=True))
            ds.append(pltpu.async_copy(ones_v,
                                       cnts_sh.at[lidx4.at[step, j]],
                                       sem_sco, add=True))
        return ds

    def wait_all(ds):
        for d in ds:
            d.wait()

    scs = []
    for k in range(2):
        lds[k][0].wait(); lds[k][1].wait()
        scs += fire_scatter(k, HB // CK)

    @pl.when(sub < NS - 1)
    def _steps_23():
        lds[2][0].wait(); lds[2][1].wait()
        sc2 = fire_scatter(2, HB // CK)
        lds[3][0].wait(); lds[3][1].wait()
        sc3 = fire_scatter(3, HB // CK)
        wait_all(sc2)
        wait_all(sc3)

    @pl.when(sub == NS - 1)
    def _step_2t():
        lds[2][0].wait(); lds[2][1].wait()
        lds[3][0].wait(); lds[3][1].wait()
        sct = [pltpu.async_copy(rows4.at[2, pl.ds(CK, CK)],
                                sums_sh.at[lidx4.at[2, 1]],
                                sem_scr, add=True),
               pltpu.async_copy(ones_v, cnts_sh.at[lidx4.at[2, 1]],
                                sem_sco, add=True)]
        wait_all(sct)

    wait_all(scs)
    plsc.subcore_barrier()

    dcr = pltpu.async_copy(cnts_sh.at[pl.ds(rbase, RT)], z16_v, sem_rd)

    def read_q(h, rows):
        return pltpu.async_copy(sums_sh.at[pl.ds(rbase + h * HB, rows)],
                                rows4.at[h, pl.ds(0, rows)], sem_ldr)

    def scale_q(h, rows):
        def body(r4, _):
            for k in range(4):
                r = r4 * 4 + k
                c = z16_v[h * HB + r, :]
                s_v = jnp.where(c > 0, 1.0 / c, 0.0)
                for g in range(DH // L):
                    rows4[h, r, pl.ds(g * L, L)] = \
                        rows4[h, r, pl.ds(g * L, L)] * s_v
            return 0
        lax.fori_loop(0, rows // 4, body, 0)

    def write_q(h, rows):
        return pltpu.async_copy(
            rows4.at[h, pl.ds(0, rows)],
            out_hbm.at[pl.ds(rbase + h * HB, rows), pl.ds(cb, DH)], sem_wr)

    rds = [read_q(k, HB) for k in range(4)]
    dcr.wait()

    rds[0].wait()
    scale_q(0, HB)
    wr0 = write_q(0, HB)
    rds[1].wait()
    scale_q(1, HB)
    wr1 = write_q(1, HB)

    @pl.when(sub < NS - 1)
    def _out_full():
        rds[2].wait()
        scale_q(2, HB)
        wr2 = write_q(2, HB)
        rds[3].wait()
        scale_q(3, HB)
        wr3 = write_q(3, HB)
        wr2.wait(); wr3.wait()

    @pl.when(sub == NS - 1)
    def _out_short():
        rds[2].wait()
        rds[3].wait()
        scale_q(2, CK)
        wrt = write_q(2, CK)
        wrt.wait()

    wr0.wait(); wr1.wait()


_agg = pl.kernel(
    _body,
    out_type=jax.ShapeDtypeStruct((N, D), jnp.float32),
    mesh=plsc.VectorSubcoreMesh(core_axis_name="c", subcore_axis_name="s",
                                num_cores=NC, num_subcores=NS),
    compiler_params=pltpu.CompilerParams(use_tc_tiling_on_sc=False),
    scratch_types=[
        pltpu.VMEM_SHARED((NHP, DH), jnp.float32),
        pltpu.VMEM_SHARED((NHP, L), jnp.float32),
        pltpu.VMEM((4, HB, DH), jnp.float32),
        pltpu.VMEM((4, HB // CK, CK), jnp.int32),
        pltpu.VMEM((CK, L), jnp.float32),
        pltpu.VMEM((RT, L), jnp.float32),
        pltpu.VMEM((HB, DH), jnp.float32),
        pltpu.SemaphoreType.DMA,
        pltpu.SemaphoreType.DMA,
        pltpu.SemaphoreType.DMA,
        pltpu.SemaphoreType.DMA,
        pltpu.SemaphoreType.DMA,
        pltpu.SemaphoreType.DMA,
        pltpu.SemaphoreType.DMA,
        pltpu.SemaphoreType.DMA,
    ],
)


@jax.jit
def kernel(M, nodes):
    nodes2 = nodes.astype(jnp.int32).reshape(NUM_MSG // CK, CK)
    return _agg(M, nodes2)


# speedup vs baseline: 4.4353x; 4.4353x over previous
import jax
import jax.numpy as jnp
from jax import lax
from jax.experimental import pallas as pl
from jax.experimental.pallas import tpu as pltpu
from jax.experimental.pallas import tpu_sc as plsc

N = 10000
D = 128
DH = 64
NUM_MSG = 10000
NC = 2
NS = 16
L = 16

HB = 160
CK = 80
MPT = 640
NHP = 10240
RT = 640


def _body(m_hbm, nodes2_hbm, zeros_hbm, out_hbm,
          sums_sh, cnts_sh,
          rows4, lidx4, ones_v, z16_v,
          sem_ldn, sem_ldr, sem_scr, sem_sco, sem_wr, sem_rd, sem_zs, sem_zc):
    core = lax.axis_index("c")
    sub = lax.axis_index("s")
    zvec = jnp.zeros((L,), jnp.float32)
    onevec = jnp.ones((L,), jnp.float32)
    cb = core * DH
    rbase = sub * RT
    mbase = sub * MPT
    bbase = sub * (MPT // CK)

    def fire_load(step):
        mb = jnp.minimum(mbase + step * HB, NUM_MSG - HB)
        blk = jnp.minimum(bbase + step * (HB // CK),
                          NUM_MSG // CK - HB // CK)
        dn = pltpu.async_copy(nodes2_hbm.at[pl.ds(blk, HB // CK)],
                              lidx4.at[step], sem_ldn)
        dr = pltpu.async_copy(m_hbm.at[pl.ds(mb, HB), pl.ds(cb, DH)],
                              rows4.at[step], sem_ldr)
        return dn, dr

    lds = [fire_load(0), fire_load(1), fire_load(2), fire_load(3)]

    for i in range(CK):
        ones_v[i, :] = onevec

    dz = pltpu.async_copy(zeros_hbm, sums_sh.at[pl.ds(rbase, RT)], sem_zs)
    dc = pltpu.async_copy(zeros_hbm.at[:, pl.ds(0, L)],
                          cnts_sh.at[pl.ds(rbase, RT)], sem_zc)
    dz.wait()
    dc.wait()
    plsc.subcore_barrier()

    def fire_scatter(step, nchunks):
        ds = []
        for j in range(nchunks):
            ds.append(pltpu.async_copy(rows4.at[step, pl.ds(j * CK, CK)],
                                       sums_sh.at[lidx4.at[step, j]],
                                       sem_scr, add=True))
            ds.append(pltpu.async_copy(ones_v,
                                       cnts_sh.at[lidx4.at[step, j]],
                                       sem_sco, add=True))
        return ds

    def wait_all(ds):
        for d in ds:
            d.wait()

    scs = []
    for k in range(2):
        lds[k][0].wait(); lds[k][1].wait()
        scs += fire_scatter(k, HB // CK)

    @pl.when(sub < NS - 1)
    def _steps_23():
        lds[2][0].wait(); lds[2][1].wait()
        sc2 = fire_scatter(2, HB // CK)
        lds[3][0].wait(); lds[3][1].wait()
        sc3 = fire_scatter(3, HB // CK)
        wait_all(sc2)
        wait_all(sc3)

    @pl.when(sub == NS - 1)
    def _step_2t():
        lds[2][0].wait(); lds[2][1].wait()
        lds[3][0].wait(); lds[3][1].wait()
        sct = [pltpu.async_copy(rows4.at[2, pl.ds(CK, CK)],
                                sums_sh.at[lidx4.at[2, 1]],
                                sem_scr, add=True),
               pltpu.async_copy(ones_v, cnts_sh.at[lidx4.at[2, 1]],
                                sem_sco, add=True)]
        wait_all(sct)

    wait_all(scs)
    plsc.subcore_barrier()

    dcr = pltpu.async_copy(cnts_sh.at[pl.ds(rbase, RT)], z16_v, sem_rd)

    def read_q(h, rows):
        return pltpu.async_copy(sums_sh.at[pl.ds(rbase + h * HB, rows)],
                                rows4.at[h, pl.ds(0, rows)], sem_ldr)

    def scale_q(h, rows):
        def body(r4, _):
            for k in range(4):
                r = r4 * 4 + k
                c = z16_v[h * HB + r, :]
                s_v = jnp.where(c > 0, 1.0 / c, 0.0)
                for g in range(DH // L):
                    rows4[h, r, pl.ds(g * L, L)] = \
                        rows4[h, r, pl.ds(g * L, L)] * s_v
            return 0
        lax.fori_loop(0, rows // 4, body, 0)

    def write_q(h, rows):
        return pltpu.async_copy(
            rows4.at[h, pl.ds(0, rows)],
            out_hbm.at[pl.ds(rbase + h * HB, rows), pl.ds(cb, DH)], sem_wr)

    rds = [read_q(k, HB) for k in range(4)]
    dcr.wait()

    rds[0].wait()
    scale_q(0, HB)
    wr0 = write_q(0, HB)
    rds[1].wait()
    scale_q(1, HB)
    wr1 = write_q(1, HB)

    @pl.when(sub < NS - 1)
    def _out_full():
        rds[2].wait()
        scale_q(2, HB)
        wr2 = write_q(2, HB)
        rds[3].wait()
        scale_q(3, HB)
        wr3 = write_q(3, HB)
        wr2.wait(); wr3.wait()

    @pl.when(sub == NS - 1)
    def _out_short():
        rds[2].wait()
        rds[3].wait()
        scale_q(2, CK)
        wrt = write_q(2, CK)
        wrt.wait()

    wr0.wait(); wr1.wait()


_agg = pl.kernel(
    _body,
    out_type=jax.ShapeDtypeStruct((N, D), jnp.float32),
    mesh=plsc.VectorSubcoreMesh(core_axis_name="c", subcore_axis_name="s",
                                num_cores=NC, num_subcores=NS),
    compiler_params=pltpu.CompilerParams(use_tc_tiling_on_sc=False),
    scratch_types=[
        pltpu.VMEM_SHARED((NHP, DH), jnp.float32),
        pltpu.VMEM_SHARED((NHP, L), jnp.float32),
        pltpu.VMEM((4, HB, DH), jnp.float32),
        pltpu.VMEM((4, HB // CK, CK), jnp.int32),
        pltpu.VMEM((CK, L), jnp.float32),
        pltpu.VMEM((RT, L), jnp.float32),
        pltpu.SemaphoreType.DMA,
        pltpu.SemaphoreType.DMA,
        pltpu.SemaphoreType.DMA,
        pltpu.SemaphoreType.DMA,
        pltpu.SemaphoreType.DMA,
        pltpu.SemaphoreType.DMA,
        pltpu.SemaphoreType.DMA,
        pltpu.SemaphoreType.DMA,
    ],
)


@jax.jit
def kernel(M, nodes):
    zeros = jnp.zeros((RT, DH), jnp.float32)
    nodes2 = nodes.astype(jnp.int32).reshape(NUM_MSG // CK, CK)
    return _agg(M, nodes2, zeros)
